# Initial kernel scaffold; baseline (speedup 1.0000x reference)
import jax
import jax.numpy as jnp
from jax.experimental import pallas as pl


def kernel(inputs, day_table):
    raise NotImplementedError("write your pallas kernel here")


# speedup vs baseline: 3.3709x; 3.3709x over previous
import functools

import jax
import jax.numpy as jnp
from jax import lax
from jax.experimental import pallas as pl
from jax.experimental.pallas import tpu as pltpu
from jax.experimental.pallas import tpu_sc as plsc

_NC = 2
_NS = 16
_NW = _NC * _NS
_L = 16


@functools.lru_cache(maxsize=None)
def _build(n_idx: int, n_vocab: int, n_dim: int, stride: int):
    assert n_idx % _NW == 0
    per_w = n_idx // _NW
    nb = 4096
    while per_w % nb:
        nb //= 2
    n_blk = per_w // nb
    in_blk = nb * stride
    out_blk = nb * n_dim
    tab_words = n_vocab * n_dim
    vregs_per_blk = out_blk // _L
    idx_per_vreg = _L // n_dim
    assert n_dim & (n_dim - 1) == 0, "n_dim must be a power of two"
    dim_sh = n_dim.bit_length() - 1

    mesh = plsc.VectorSubcoreMesh(core_axis_name="c", subcore_axis_name="s")

    @functools.partial(
        pl.kernel,
        out_type=jax.ShapeDtypeStruct((n_idx * n_dim,), jnp.float32),
        mesh=mesh,
        compiler_params=pltpu.CompilerParams(needs_layout_passes=False),
        scratch_types=[
            pltpu.VMEM((tab_words,), jnp.float32),
            pltpu.VMEM((in_blk,), jnp.int32),
            pltpu.VMEM((in_blk,), jnp.int32),
            pltpu.VMEM((out_blk,), jnp.float32),
            pltpu.VMEM((out_blk,), jnp.float32),
            pltpu.SemaphoreType.DMA,
            pltpu.SemaphoreType.DMA,
            pltpu.SemaphoreType.DMA,
            pltpu.SemaphoreType.DMA,
            pltpu.SemaphoreType.DMA,
        ],
    )
    def emb(in_hbm, tab_hbm, out_hbm, tab_v, in_a, in_b, out_a, out_b,
            sem_t, sem_i0, sem_i1, sem_o0, sem_o1):
        wid = lax.axis_index("s") * _NC + lax.axis_index("c")
        base_in = wid * (per_w * stride)
        base_out = wid * (per_w * n_dim)

        pltpu.async_copy(tab_hbm, tab_v, sem_t).wait()

        lane = lax.iota(jnp.int32, _L)
        pat_in = lax.shift_right_logical(lane, dim_sh) * stride
        pat_col = lax.bitwise_and(lane, n_dim - 1)

        in_bufs = (in_a, in_b)
        out_bufs = (out_a, out_b)
        in_sems = (sem_i0, sem_i1)
        out_sems = (sem_o0, sem_o1)

        def start_in(g):
            return pltpu.async_copy(
                in_hbm.at[pl.ds(base_in + g * in_blk, in_blk)],
                in_bufs[g % 2], in_sems[g % 2])

        def start_out(g):
            return pltpu.async_copy(
                out_bufs[g % 2],
                out_hbm.at[pl.ds(base_out + g * out_blk, out_blk)],
                out_sems[g % 2])

        def compute(g):
            src = in_bufs[g % 2]
            dst = out_bufs[g % 2]

            def body(v, _):
                a1 = pat_in + v * (idx_per_vreg * stride)
                rows = plsc.load_gather(src, [a1])
                a2 = lax.shift_left(rows, dim_sh) + pat_col
                dst[pl.ds(v * _L, _L)] = plsc.load_gather(tab_v, [a2])
                return 0

            lax.fori_loop(0, vregs_per_blk, body, 0, unroll=4)

        in_copies = [None] * n_blk
        out_copies = [None] * n_blk
        in_copies[0] = start_in(0)
        for g in range(n_blk):
            if g + 1 < n_blk:
                in_copies[g + 1] = start_in(g + 1)
            in_copies[g].wait()
            if g >= 2:
                out_copies[g - 2].wait()
            compute(g)
            out_copies[g] = start_out(g)
        for g in range(max(0, n_blk - 2), n_blk):
            out_copies[g].wait()

    return emb


def kernel(inputs, day_table):
    b, t, c = inputs.shape
    n_vocab, n_dim = day_table.shape
    n_idx = b * t
    flat_in = inputs.reshape(-1).astype(jnp.int32)
    flat_tab = day_table.reshape(-1).astype(jnp.float32)
    out = _build(n_idx, n_vocab, n_dim, c)(flat_in, flat_tab)
    return out.reshape(b, t, n_dim)


# speedup vs baseline: 81.8911x; 24.2936x over previous
import functools

import jax
import jax.numpy as jnp
from jax import lax
from jax.experimental import pallas as pl
from jax.experimental.pallas import tpu as pltpu
from jax.experimental.pallas import tpu_sc as plsc

_NC = 2
_NS = 16
_NW = _NC * _NS
_L = 16
_BL = 128


@functools.lru_cache(maxsize=None)
def _build(n_panels: int, stride: int, n_vocab: int, n_dim: int):
    assert n_panels % _NW == 0
    per_w = n_panels // _NW
    pb = 40
    while per_w % pb:
        pb -= 1
    n_blk = per_w // pb
    tab_words = n_vocab * n_dim
    assert n_dim & (n_dim - 1) == 0
    dim_sh = n_dim.bit_length() - 1
    groups = _BL // _L

    mesh = plsc.VectorSubcoreMesh(core_axis_name="c", subcore_axis_name="s")

    @functools.partial(
        pl.kernel,
        out_type=jax.ShapeDtypeStruct((n_panels, n_dim, _BL), jnp.float32),
        mesh=mesh,
        compiler_params=pltpu.CompilerParams(needs_layout_passes=False),
        scratch_types=[
            pltpu.VMEM((tab_words,), jnp.float32),
            pltpu.VMEM((pb, stride, _BL), jnp.int32),
            pltpu.VMEM((pb, stride, _BL), jnp.int32),
            pltpu.VMEM((pb, n_dim, _BL), jnp.float32),
            pltpu.VMEM((pb, n_dim, _BL), jnp.float32),
            pltpu.SemaphoreType.DMA,
            pltpu.SemaphoreType.DMA,
            pltpu.SemaphoreType.DMA,
            pltpu.SemaphoreType.DMA,
            pltpu.SemaphoreType.DMA,
        ],
    )
    def emb(in_hbm, tab_hbm, out_hbm, tab_v, in_a, in_b, out_a, out_b,
            sem_t, sem_i0, sem_i1, sem_o0, sem_o1):
        wid = lax.axis_index("s") * _NC + lax.axis_index("c")
        p0 = wid * per_w

        pltpu.async_copy(tab_hbm, tab_v, sem_t).wait()

        in_bufs = (in_a, in_b)
        out_bufs = (out_a, out_b)
        in_sems = (sem_i0, sem_i1)
        out_sems = (sem_o0, sem_o1)

        def start_in(g):
            return pltpu.async_copy(
                in_hbm.at[pl.ds(p0 + g * pb, pb)],
                in_bufs[g % 2], in_sems[g % 2])

        def start_out(g):
            return pltpu.async_copy(
                out_bufs[g % 2],
                out_hbm.at[pl.ds(p0 + g * pb, pb)],
                out_sems[g % 2])

        def compute(g):
            src = in_bufs[g % 2]
            dst = out_bufs[g % 2]

            def panel_body(p, _):
                def group_body(j, _):
                    idx = src[p, 0, pl.ds(j * _L, _L)]
                    a = lax.shift_left(idx, dim_sh)
                    for c in range(n_dim):
                        dst[p, c, pl.ds(j * _L, _L)] = (
                            plsc.load_gather(tab_v, [a + c]))
                    return 0

                lax.fori_loop(0, groups, group_body, 0, unroll=2)
                return 0

            lax.fori_loop(0, pb, panel_body, 0)

        in_copies = [None] * n_blk
        out_copies = [None] * n_blk
        in_copies[0] = start_in(0)
        for g in range(n_blk):
            if g + 1 < n_blk:
                in_copies[g + 1] = start_in(g + 1)
            in_copies[g].wait()
            if g >= 2:
                out_copies[g - 2].wait()
            compute(g)
            out_copies[g] = start_out(g)
        for g in range(max(0, n_blk - 2), n_blk):
            out_copies[g].wait()

    return emb


def kernel(inputs, day_table):
    b, t, c = inputs.shape
    n_vocab, n_dim = day_table.shape
    assert b % _BL == 0
    nbt = b // _BL
    n_panels = t * nbt
    x = (inputs.astype(jnp.int32)
         .reshape(nbt, _BL, t, c)
         .transpose(2, 0, 3, 1)
         .reshape(n_panels, c, _BL))
    flat_tab = day_table.reshape(-1).astype(jnp.float32)
    o = _build(n_panels, c, n_vocab, n_dim)(x, flat_tab)
    return (o.reshape(t, nbt, n_dim, _BL)
            .transpose(1, 3, 0, 2)
            .reshape(b, t, n_dim))


# speedup vs baseline: 84.6733x; 1.0340x over previous
import functools

import jax
import jax.numpy as jnp
from jax import lax
from jax.experimental import pallas as pl
from jax.experimental.pallas import tpu as pltpu
from jax.experimental.pallas import tpu_sc as plsc

_NC = 2
_NS = 16
_NW = _NC * _NS
_L = 16
_BL = 128


@functools.lru_cache(maxsize=None)
def _build(n_panels: int, stride: int, n_vocab: int, n_dim: int):
    assert n_panels % _NW == 0
    per_w = n_panels // _NW
    pb = 80
    while per_w % pb:
        pb -= 1
    n_blk = per_w // pb
    tab_words = n_vocab * n_dim
    assert n_dim & (n_dim - 1) == 0
    dim_sh = n_dim.bit_length() - 1
    groups = _BL // _L

    mesh = plsc.VectorSubcoreMesh(core_axis_name="c", subcore_axis_name="s")

    @functools.partial(
        pl.kernel,
        out_type=jax.ShapeDtypeStruct((n_panels, n_dim, _BL), jnp.float32),
        mesh=mesh,
        compiler_params=pltpu.CompilerParams(needs_layout_passes=False),
        scratch_types=[
            pltpu.VMEM((tab_words,), jnp.float32),
            pltpu.VMEM((pb, 1, _BL), jnp.int32),
            pltpu.VMEM((pb, 1, _BL), jnp.int32),
            pltpu.VMEM((pb, n_dim, _BL), jnp.float32),
            pltpu.VMEM((pb, n_dim, _BL), jnp.float32),
            pltpu.SemaphoreType.DMA,
            pltpu.SemaphoreType.DMA,
            pltpu.SemaphoreType.DMA,
            pltpu.SemaphoreType.DMA,
            pltpu.SemaphoreType.DMA,
        ],
    )
    def emb(in_hbm, tab_hbm, out_hbm, tab_v, in_a, in_b, out_a, out_b,
            sem_t, sem_i0, sem_i1, sem_o0, sem_o1):
        wid = lax.axis_index("s") * _NC + lax.axis_index("c")
        p0 = wid * per_w

        pltpu.async_copy(tab_hbm, tab_v, sem_t).wait()

        in_bufs = (in_a, in_b)
        out_bufs = (out_a, out_b)
        in_sems = (sem_i0, sem_i1)
        out_sems = (sem_o0, sem_o1)

        def start_in(g):
            return pltpu.async_copy(
                in_hbm.at[pl.ds(p0 + g * pb, pb), pl.ds(0, 1)],
                in_bufs[g % 2], in_sems[g % 2])

        def start_out(g):
            return pltpu.async_copy(
                out_bufs[g % 2],
                out_hbm.at[pl.ds(p0 + g * pb, pb)],
                out_sems[g % 2])

        def compute(g):
            src = in_bufs[g % 2]
            dst = out_bufs[g % 2]

            def panel_body(p, _):
                def group_body(j, _):
                    idx = src[p, 0, pl.ds(j * _L, _L)]
                    a = lax.shift_left(idx, dim_sh)
                    for c in range(n_dim):
                        dst[p, c, pl.ds(j * _L, _L)] = (
                            plsc.load_gather(tab_v, [a + c]))
                    return 0

                lax.fori_loop(0, groups, group_body, 0, unroll=2)
                return 0

            lax.fori_loop(0, pb, panel_body, 0)

        in_copies = [None] * n_blk
        out_copies = [None] * n_blk
        in_copies[0] = start_in(0)
        for g in range(n_blk):
            if g + 1 < n_blk:
                in_copies[g + 1] = start_in(g + 1)
            in_copies[g].wait()
            if g >= 2:
                out_copies[g - 2].wait()
            compute(g)
            out_copies[g] = start_out(g)
        for g in range(max(0, n_blk - 2), n_blk):
            out_copies[g].wait()

    return emb


def kernel(inputs, day_table):
    b, t, c = inputs.shape
    n_vocab, n_dim = day_table.shape
    assert b % _BL == 0
    nbt = b // _BL
    n_panels = t * nbt
    x = (inputs.astype(jnp.int32)
         .reshape(nbt, _BL, t, c)
         .transpose(2, 0, 3, 1)
         .reshape(n_panels, c, _BL))
    flat_tab = day_table.reshape(-1).astype(jnp.float32)
    o = _build(n_panels, c, n_vocab, n_dim)(x, flat_tab)
    return (o.reshape(t, nbt, n_dim, _BL)
            .transpose(1, 3, 0, 2)
            .reshape(b, t, n_dim))
